# baseline (speedup 1.0000x reference)
import functools
import jax
import jax.numpy as jnp
from jax import lax
from jax.experimental import pallas as pl
from jax.experimental.pallas import tpu as pltpu
from jax.experimental.pallas import tpu_sc as plsc

EPS = 1e-06
CHI2 = 9.0
DIST_SCALE = 3.0


def _normalize_rows(x, eps=EPS):
    n = jnp.linalg.norm(x, axis=1, keepdims=True)
    return x / jnp.maximum(n, eps)


def _skew_rows(k):
    kx, ky, kz = k[:, 0], k[:, 1], k[:, 2]
    O = jnp.zeros_like(kx)
    row0 = jnp.stack([O, -kz, ky], axis=1)
    row1 = jnp.stack([kz, O, -kx], axis=1)
    row2 = jnp.stack([-ky, kx, O], axis=1)
    return jnp.stack([row0, row1, row2], axis=1)


def _align_rotation(g_src, g_tgt, eps=EPS):
    u = _normalize_rows(g_src, eps)
    v = _normalize_rows(g_tgt, eps)
    axis = jnp.cross(u, v)
    axis_norm = jnp.linalg.norm(axis, axis=1, keepdims=True)
    dot = jnp.clip(jnp.sum(u * v, axis=1, keepdims=True), -1.0, 1.0)
    parallel = axis_norm < 1e-06
    k = axis / (axis_norm + eps)
    theta = jnp.arccos(dot)
    sin_t = jnp.sin(theta)
    cos_t = jnp.cos(theta)
    K = _skew_rows(k)
    I = jnp.eye(3, dtype=g_src.dtype)[None]
    R = I + sin_t[:, :, None] * K + (1.0 - cos_t)[:, :, None] * (K @ K)
    antipar = parallel & (dot < 0.0)
    B = u.shape[0]
    ex = jnp.tile(jnp.array([1.0, 0.0, 0.0], dtype=u.dtype)[None, :], (B, 1))
    ey = jnp.tile(jnp.array([0.0, 1.0, 0.0], dtype=u.dtype)[None, :], (B, 1))
    use_ex = jnp.abs(u[:, 0:1]) < 0.9
    basis = jnp.where(use_ex, ex, ey)
    axis2 = _normalize_rows(jnp.cross(u, basis), eps)
    K2 = _skew_rows(axis2)
    R_anti = I + 2.0 * (K2 @ K2)
    R = jnp.where(antipar[:, :, None], R_anti, R)
    R = jnp.where((parallel & (dot > 0.0))[:, :, None], I, R)
    return R


def _ght_tc_kernel(src_ref, tgt_ref, srcn_ref, tgtn_ref,
                   r9_ref, c3_ref, gq_ref, xxr_ref,
                   idxp_ref, idxq_ref, incp_ref, incq_ref,
                   geomp_ref, geomq_ref, *, n):
    f32 = jnp.float32
    bf16 = jnp.bfloat16
    N = n

    R3 = jnp.concatenate(
        [jnp.concatenate([jnp.full((1, 1), r9_ref[0, 0, 3 * i + j], f32)
                          for j in range(3)], axis=1) for i in range(3)],
        axis=0)
    R3_bf = R3.astype(bf16)
    c = jnp.concatenate([jnp.full((1, 1), c3_ref[0, 0, i], f32)
                         for i in range(3)], axis=0)
    gqx, gqy, gqz = gq_ref[0, 0, 0], gq_ref[0, 0, 1], gq_ref[0, 0, 2]

    mmdims = (((1,), (0,)), ((), ()))
    cdims = (((0,), (0,)), ((), ()))

    src3 = src_ref[0]
    tgt3 = tgt_ref[0]
    src_rot = lax.dot_general(R3_bf, src3.astype(bf16), mmdims,
                              preferred_element_type=f32)
    si3 = src_rot + c

    mm = lax.dot_general(si3.astype(bf16), tgt3.astype(bf16), cdims,
                         preferred_element_type=f32)
    inner = -2.0 * mm
    t0, t1, t2 = tgt3[0:1, :], tgt3[1:2, :], tgt3[2:3, :]
    xx_row = xxr_ref[0]
    yy_row = t0 * t0 + t1 * t1 + t2 * t2
    dist = (jnp.transpose(xx_row) + inner) + yy_row

    iota0 = lax.broadcasted_iota(jnp.int32, (N, N), 0)
    iota1 = lax.broadcasted_iota(jnp.int32, (N, N), 1)
    BIGI = jnp.int32(N)

    minp_col = jnp.min(dist, axis=1, keepdims=True)
    idxp_col = jnp.min(jnp.where(dist == minp_col, iota1, BIGI), axis=1,
                       keepdims=True)
    minq_row = jnp.min(dist, axis=0, keepdims=True)
    idxq_row = jnp.min(jnp.where(dist == minq_row, iota0, BIGI), axis=0,
                       keepdims=True)

    tn3 = tgtn_ref[0]
    sn3 = srcn_ref[0]
    snr3 = lax.dot_general(R3_bf, sn3.astype(bf16), mmdims,
                           preferred_element_type=f32)
    inc_p_row = (snr3[0:1, :] * gqx + snr3[1:2, :] * gqy + snr3[2:3, :] * gqz)
    inc_q_row = (tn3[0:1, :] * gqx + tn3[1:2, :] * gqy + tn3[2:3, :] * gqz)

    minp_row = jnp.transpose(minp_col)
    nnp_row = jnp.sqrt(jnp.maximum(minp_row, 0.0) + EPS)
    nnq_row = jnp.sqrt(jnp.maximum(minq_row, 0.0) + EPS)

    K_RANK = (N - 1) // 2 + 1
    C = N // 128

    def rank_select(m_row):
        u = lax.bitcast_convert_type(m_row, jnp.int32).reshape(C, 128)
        res = jnp.int32(0)
        for bit in range(30, -1, -1):
            trial = res | jnp.int32(1 << bit)
            cnt = jnp.sum((u >= trial).astype(jnp.int32))
            res = jnp.where(cnt >= N - K_RANK + 1, trial, res)
        return lax.bitcast_convert_type(res, f32)

    mp_row = jnp.maximum(minp_row, 0.0) + EPS
    mq_row = jnp.maximum(minq_row, 0.0) + EPS
    med_p = jnp.sqrt(rank_select(mp_row))
    med_q = jnp.sqrt(rank_select(mq_row))

    geom_p = (nnp_row <= DIST_SCALE * med_p).astype(f32)
    geom_q = (nnq_row <= DIST_SCALE * med_q).astype(f32)

    b = pl.program_id(0)
    off = (b * N).astype(jnp.int32)
    idxp_ref[0, :, :] = jnp.transpose(idxp_col) + off
    idxq_ref[0, :, :] = idxq_row + off
    incp_ref[0, :, :] = inc_p_row
    incq_ref[0, :, :] = inc_q_row
    geomp_ref[0, :, :] = geom_p
    geomq_ref[0, :, :] = geom_q


def _make_sc_gate(BN):
    f32 = jnp.float32
    info = plsc.get_sparse_core_info()
    NC, NS, L = info.num_cores, info.num_subcores, info.num_lanes
    NW = NC * NS
    per_w = BN // NW
    CHUNK = 128
    n_chunks = per_w // CHUNK
    mesh = plsc.VectorSubcoreMesh(core_axis_name="c", subcore_axis_name="s")

    @functools.partial(
        pl.kernel, mesh=mesh,
        out_type=[jax.ShapeDtypeStruct((BN,), f32),
                  jax.ShapeDtypeStruct((BN,), f32)],
        scratch_types=[
            pltpu.VMEM((CHUNK,), jnp.int32),
            pltpu.VMEM((CHUNK,), f32),
            pltpu.VMEM((per_w,), f32),
            pltpu.VMEM((per_w,), f32),
            pltpu.VMEM((per_w,), f32),
            pltpu.VMEM((per_w,), f32),
            pltpu.SemaphoreType.DMA,
        ],
    )
    def sc_gate(incp_hbm, incq_hbm, gidxp_hbm, gidxq_hbm,
                geomp_hbm, geomq_hbm, kef_hbm,
                wp_hbm, wq_hbm,
                idx_v, val_v, inc_v, geom_v, kef_v, out_v, sem):
        wid = lax.axis_index("s") * NC + lax.axis_index("c")
        base = wid * per_w
        pltpu.sync_copy(kef_hbm.at[pl.ds(base, per_w)], kef_v)

        def side(table_hbm, gidx_hbm, inc_hbm, geom_hbm, w_hbm):
            pltpu.sync_copy(inc_hbm.at[pl.ds(base, per_w)], inc_v)
            pltpu.sync_copy(geom_hbm.at[pl.ds(base, per_w)], geom_v)
            for ci in range(n_chunks):
                o = ci * CHUNK
                pltpu.sync_copy(gidx_hbm.at[pl.ds(base + o, CHUNK)], idx_v)
                pltpu.async_copy(table_hbm.at[idx_v], val_v, sem).wait()
                for v in range(CHUNK // L):
                    s = v * L
                    inc = inc_v[pl.ds(o + s, L)]
                    ref = val_v[pl.ds(s, L)]
                    kef = kef_v[pl.ds(o + s, L)]
                    geom = geom_v[pl.ds(o + s, L)]
                    d = inc - ref
                    w = geom / (1.0 + jnp.exp(kef * d * d - CHI2))
                    out_v[pl.ds(o + s, L)] = w
            pltpu.sync_copy(out_v, w_hbm.at[pl.ds(base, per_w)])

        side(incq_hbm, gidxp_hbm, incp_hbm, geomp_hbm, wp_hbm)
        side(incp_hbm, gidxq_hbm, incq_hbm, geomq_hbm, wq_hbm)

    return sc_gate


@jax.jit
def kernel(src, tgt, src_n, tgt_n, g_p, k_p, g_q, k_q):
    B, _, N = src.shape
    f32 = jnp.float32

    Rg = _align_rotation(g_p, g_q)
    src_rot_o = jnp.matmul(Rg, src)
    t_center = (jnp.mean(tgt, axis=2, keepdims=True)
                - jnp.mean(src_rot_o, axis=2, keepdims=True))
    src_init_o = src_rot_o + t_center
    xx_o = jnp.sum(src_init_o ** 2, axis=1, keepdims=True)
    k_eff = k_p * k_q / (k_p + k_q + EPS)

    vec_spec = pl.BlockSpec((1, 3, N), lambda b: (b, 0, 0))
    row_spec = pl.BlockSpec((1, 1, N), lambda b: (b, 0, 0))
    smem9 = pl.BlockSpec((1, 1, 9), lambda b: (b, 0, 0), memory_space=pltpu.SMEM)
    smem3 = pl.BlockSpec((1, 1, 3), lambda b: (b, 0, 0), memory_space=pltpu.SMEM)

    outs = pl.pallas_call(
        functools.partial(_ght_tc_kernel, n=N),
        grid=(B,),
        in_specs=[vec_spec, vec_spec, vec_spec, vec_spec,
                  smem9, smem3, smem3, row_spec],
        out_specs=[row_spec] * 6,
        out_shape=[jax.ShapeDtypeStruct((B, 1, N), jnp.int32),
                   jax.ShapeDtypeStruct((B, 1, N), jnp.int32),
                   jax.ShapeDtypeStruct((B, 1, N), f32),
                   jax.ShapeDtypeStruct((B, 1, N), f32),
                   jax.ShapeDtypeStruct((B, 1, N), f32),
                   jax.ShapeDtypeStruct((B, 1, N), f32)],
        compiler_params=pltpu.CompilerParams(
            dimension_semantics=("arbitrary",),
        ),
    )(src, tgt, src_n, tgt_n,
      Rg.reshape(B, 1, 9), t_center.reshape(B, 1, 3),
      g_q.reshape(B, 1, 3), xx_o)
    gidxp, gidxq, incp, incq, geomp, geomq = outs

    BN = B * N
    kef_flat = jnp.broadcast_to(k_eff.reshape(B, 1), (B, N)).reshape(BN)
    sc_gate = _make_sc_gate(BN)
    wp_flat, wq_flat = sc_gate(
        incp.reshape(BN), incq.reshape(BN),
        gidxp.reshape(BN), gidxq.reshape(BN),
        geomp.reshape(BN), geomq.reshape(BN), kef_flat)
    return (wp_flat.reshape(B, 1, N), wq_flat.reshape(B, 1, N))


# speedup vs baseline: 1.5910x; 1.5910x over previous
import functools
import jax
import jax.numpy as jnp
from jax import lax
from jax.experimental import pallas as pl
from jax.experimental.pallas import tpu as pltpu

EPS = 1e-06
CHI2 = 9.0
DIST_SCALE = 3.0


def _normalize_rows(x, eps=EPS):
    n = jnp.linalg.norm(x, axis=1, keepdims=True)
    return x / jnp.maximum(n, eps)


def _skew_rows(k):
    kx, ky, kz = k[:, 0], k[:, 1], k[:, 2]
    O = jnp.zeros_like(kx)
    row0 = jnp.stack([O, -kz, ky], axis=1)
    row1 = jnp.stack([kz, O, -kx], axis=1)
    row2 = jnp.stack([-ky, kx, O], axis=1)
    return jnp.stack([row0, row1, row2], axis=1)


def _align_rotation(g_src, g_tgt, eps=EPS):
    u = _normalize_rows(g_src, eps)
    v = _normalize_rows(g_tgt, eps)
    axis = jnp.cross(u, v)
    axis_norm = jnp.linalg.norm(axis, axis=1, keepdims=True)
    dot = jnp.clip(jnp.sum(u * v, axis=1, keepdims=True), -1.0, 1.0)
    parallel = axis_norm < 1e-06
    k = axis / (axis_norm + eps)
    theta = jnp.arccos(dot)
    sin_t = jnp.sin(theta)
    cos_t = jnp.cos(theta)
    K = _skew_rows(k)
    I = jnp.eye(3, dtype=g_src.dtype)[None]
    R = I + sin_t[:, :, None] * K + (1.0 - cos_t)[:, :, None] * (K @ K)
    antipar = parallel & (dot < 0.0)
    B = u.shape[0]
    ex = jnp.tile(jnp.array([1.0, 0.0, 0.0], dtype=u.dtype)[None, :], (B, 1))
    ey = jnp.tile(jnp.array([0.0, 1.0, 0.0], dtype=u.dtype)[None, :], (B, 1))
    use_ex = jnp.abs(u[:, 0:1]) < 0.9
    basis = jnp.where(use_ex, ex, ey)
    axis2 = _normalize_rows(jnp.cross(u, basis), eps)
    K2 = _skew_rows(axis2)
    R_anti = I + 2.0 * (K2 @ K2)
    R = jnp.where(antipar[:, :, None], R_anti, R)
    R = jnp.where((parallel & (dot > 0.0))[:, :, None], I, R)
    return R


def _ght_kernel(src_ref, tgt_ref, srcn_ref, tgtn_ref,
                r9_ref, c3_ref, kk_ref, gq_ref, xxr_ref,
                wp_ref, wq_ref, *, n):
    f32 = jnp.float32
    bf16 = jnp.bfloat16
    N = n

    R3 = jnp.concatenate(
        [jnp.concatenate([jnp.full((1, 1), r9_ref[0, 0, 3 * i + j], f32)
                          for j in range(3)], axis=1) for i in range(3)],
        axis=0)
    R3_bf = R3.astype(bf16)
    c = jnp.concatenate([jnp.full((1, 1), c3_ref[0, 0, i], f32)
                         for i in range(3)], axis=0)
    k_eff = kk_ref[0, 0, 0]
    gqx, gqy, gqz = gq_ref[0, 0, 0], gq_ref[0, 0, 1], gq_ref[0, 0, 2]

    mmdims = (((1,), (0,)), ((), ()))
    cdims = (((0,), (0,)), ((), ()))

    src3 = src_ref[0]
    tgt3 = tgt_ref[0]
    src_rot = lax.dot_general(R3_bf, src3.astype(bf16), mmdims,
                              preferred_element_type=f32)
    si3 = src_rot + c

    mm = lax.dot_general(si3.astype(bf16), tgt3.astype(bf16), cdims,
                         preferred_element_type=f32)
    inner = -2.0 * mm
    t0, t1, t2 = tgt3[0:1, :], tgt3[1:2, :], tgt3[2:3, :]
    xx_row = xxr_ref[0]
    yy_row = t0 * t0 + t1 * t1 + t2 * t2
    dist = (jnp.transpose(xx_row) + inner) + yy_row

    iota0 = lax.broadcasted_iota(jnp.int32, (N, N), 0)
    iota1 = lax.broadcasted_iota(jnp.int32, (N, N), 1)
    BIGI = jnp.int32(N)

    minp_col = jnp.min(dist, axis=1, keepdims=True)
    idxp_col = jnp.min(jnp.where(dist == minp_col, iota1, BIGI), axis=1,
                       keepdims=True)
    minq_row = jnp.min(dist, axis=0, keepdims=True)
    idxq_row = jnp.min(jnp.where(dist == minq_row, iota0, BIGI), axis=0,
                       keepdims=True)

    tn3 = tgtn_ref[0]
    sn3 = srcn_ref[0]
    snr3 = lax.dot_general(R3_bf, sn3.astype(bf16), mmdims,
                           preferred_element_type=f32)
    inc_p_row = (snr3[0:1, :] * gqx + snr3[1:2, :] * gqy + snr3[2:3, :] * gqz)
    inc_q_row = (tn3[0:1, :] * gqx + tn3[1:2, :] * gqy + tn3[2:3, :] * gqz)

    C = N // 128

    def row_gather(vals_row, idx_row):
        er = vals_row.reshape(C, 128)
        ir = idx_row.reshape(C, 128)
        hi = ir >> 7
        lo = ir & 127
        acc = jnp.zeros((C, 128), f32)
        for c in range(C):
            bc = jnp.broadcast_to(er[c:c + 1, :], (C, 128))
            g = jnp.take_along_axis(bc, lo, axis=1)
            acc = jnp.where(hi == c, g, acc)
        return acc.reshape(1, N)

    idxp_row = jnp.transpose(idxp_col)
    inc_p_ref_row = row_gather(inc_q_row, idxp_row)
    inc_q_ref_row = row_gather(inc_p_row, idxq_row)

    minp_row = jnp.transpose(minp_col)
    nnp_row = jnp.sqrt(jnp.maximum(minp_row, 0.0) + EPS)
    nnq_row = jnp.sqrt(jnp.maximum(minq_row, 0.0) + EPS)

    K_RANK = (N - 1) // 2 + 1
    T = jnp.int32(N - K_RANK + 1)

    def _cnt(u, t):
        return jnp.sum((u >= t).astype(jnp.int32))

    def _round2(u, res, hi_bit):
        lo = hi_bit - 1
        t1 = res | jnp.int32(1 << hi_bit)
        t2 = res | jnp.int32(1 << lo)
        t3 = t1 | jnp.int32(1 << lo)
        c1, c2, c3 = _cnt(u, t1), _cnt(u, t2), _cnt(u, t3)
        hi_ok = c1 >= T
        res = jnp.where(hi_ok, t1, res)
        lo_cnt = jnp.where(hi_ok, c3, c2)
        return jnp.where(lo_cnt >= T, res | jnp.int32(1 << lo), res)

    def rank_select2(ma_row, mb_row):
        ua = lax.bitcast_convert_type(ma_row, jnp.int32).reshape(C, 128)
        ub = lax.bitcast_convert_type(mb_row, jnp.int32).reshape(C, 128)
        ta = jnp.int32(1 << 30)
        tb = jnp.int32(1 << 30)
        resa = jnp.where(_cnt(ua, ta) >= T, ta, 0)
        resb = jnp.where(_cnt(ub, tb) >= T, tb, 0)
        for hi_bit in range(29, 0, -2):
            resa = _round2(ua, resa, hi_bit)
            resb = _round2(ub, resb, hi_bit)
        return (lax.bitcast_convert_type(resa, f32),
                lax.bitcast_convert_type(resb, f32))

    mp_row = jnp.maximum(minp_row, 0.0) + EPS
    mq_row = jnp.maximum(minq_row, 0.0) + EPS
    vp, vq = rank_select2(mp_row, mq_row)
    med_p = jnp.sqrt(vp)
    med_q = jnp.sqrt(vq)

    geom_p = (nnp_row <= DIST_SCALE * med_p).astype(f32)
    geom_q = (nnq_row <= DIST_SCALE * med_q).astype(f32)

    wp = jax.nn.sigmoid(CHI2 - k_eff * (inc_p_row - inc_p_ref_row) ** 2) * geom_p
    wq = jax.nn.sigmoid(CHI2 - k_eff * (inc_q_row - inc_q_ref_row) ** 2) * geom_q
    wp_ref[0, :, :] = wp
    wq_ref[0, :, :] = wq


@jax.jit
def kernel(src, tgt, src_n, tgt_n, g_p, k_p, g_q, k_q):
    B, _, N = src.shape
    f32 = jnp.float32

    Rg = _align_rotation(g_p, g_q)
    src_rot_o = jnp.matmul(Rg, src)
    t_center = (jnp.mean(tgt, axis=2, keepdims=True)
                - jnp.mean(src_rot_o, axis=2, keepdims=True))
    src_init_o = src_rot_o + t_center
    xx_o = jnp.sum(src_init_o ** 2, axis=1, keepdims=True)
    k_eff = k_p * k_q / (k_p + k_q + EPS)

    vec_spec = pl.BlockSpec((1, 3, N), lambda b: (b, 0, 0))
    row_spec = pl.BlockSpec((1, 1, N), lambda b: (b, 0, 0))
    out_spec = pl.BlockSpec((1, 1, N), lambda b: (b, 0, 0))
    smem9 = pl.BlockSpec((1, 1, 9), lambda b: (b, 0, 0), memory_space=pltpu.SMEM)
    smem3 = pl.BlockSpec((1, 1, 3), lambda b: (b, 0, 0), memory_space=pltpu.SMEM)
    smem1 = pl.BlockSpec((1, 1, 1), lambda b: (b, 0, 0), memory_space=pltpu.SMEM)

    wp, wq = pl.pallas_call(
        functools.partial(_ght_kernel, n=N),
        grid=(B,),
        in_specs=[vec_spec, vec_spec, vec_spec, vec_spec,
                  smem9, smem3, smem1, smem3, row_spec],
        out_specs=[out_spec, out_spec],
        out_shape=[jax.ShapeDtypeStruct((B, 1, N), f32),
                   jax.ShapeDtypeStruct((B, 1, N), f32)],
        compiler_params=pltpu.CompilerParams(
            dimension_semantics=("arbitrary",),
        ),
    )(src, tgt, src_n, tgt_n,
      Rg.reshape(B, 1, 9), t_center.reshape(B, 1, 3),
      k_eff.reshape(B, 1, 1), g_q.reshape(B, 1, 3), xx_o)
    return (wp, wq)


# speedup vs baseline: 1.6503x; 1.0373x over previous
import functools
import jax
import jax.numpy as jnp
from jax import lax
from jax.experimental import pallas as pl
from jax.experimental.pallas import tpu as pltpu

EPS = 1e-06
CHI2 = 9.0
DIST_SCALE = 3.0


def _normalize_rows(x, eps=EPS):
    n = jnp.linalg.norm(x, axis=1, keepdims=True)
    return x / jnp.maximum(n, eps)


def _skew_rows(k):
    kx, ky, kz = k[:, 0], k[:, 1], k[:, 2]
    O = jnp.zeros_like(kx)
    row0 = jnp.stack([O, -kz, ky], axis=1)
    row1 = jnp.stack([kz, O, -kx], axis=1)
    row2 = jnp.stack([-ky, kx, O], axis=1)
    return jnp.stack([row0, row1, row2], axis=1)


def _align_rotation(g_src, g_tgt, eps=EPS):
    u = _normalize_rows(g_src, eps)
    v = _normalize_rows(g_tgt, eps)
    axis = jnp.cross(u, v)
    axis_norm = jnp.linalg.norm(axis, axis=1, keepdims=True)
    dot = jnp.clip(jnp.sum(u * v, axis=1, keepdims=True), -1.0, 1.0)
    parallel = axis_norm < 1e-06
    k = axis / (axis_norm + eps)
    theta = jnp.arccos(dot)
    sin_t = jnp.sin(theta)
    cos_t = jnp.cos(theta)
    K = _skew_rows(k)
    I = jnp.eye(3, dtype=g_src.dtype)[None]
    R = I + sin_t[:, :, None] * K + (1.0 - cos_t)[:, :, None] * (K @ K)
    antipar = parallel & (dot < 0.0)
    B = u.shape[0]
    ex = jnp.tile(jnp.array([1.0, 0.0, 0.0], dtype=u.dtype)[None, :], (B, 1))
    ey = jnp.tile(jnp.array([0.0, 1.0, 0.0], dtype=u.dtype)[None, :], (B, 1))
    use_ex = jnp.abs(u[:, 0:1]) < 0.9
    basis = jnp.where(use_ex, ex, ey)
    axis2 = _normalize_rows(jnp.cross(u, basis), eps)
    K2 = _skew_rows(axis2)
    R_anti = I + 2.0 * (K2 @ K2)
    R = jnp.where(antipar[:, :, None], R_anti, R)
    R = jnp.where((parallel & (dot > 0.0))[:, :, None], I, R)
    return R


def _ght_kernel(src_ref, tgt_ref, srcn_ref, tgtn_ref,
                r9_ref, c3_ref, kk_ref, gq_ref, xxr_ref,
                wp_ref, wq_ref, *, n):
    f32 = jnp.float32
    bf16 = jnp.bfloat16
    N = n

    R3 = jnp.concatenate(
        [jnp.concatenate([jnp.full((1, 1), r9_ref[0, 0, 3 * i + j], f32)
                          for j in range(3)], axis=1) for i in range(3)],
        axis=0)
    R3_bf = R3.astype(bf16)
    c = jnp.concatenate([jnp.full((1, 1), c3_ref[0, 0, i], f32)
                         for i in range(3)], axis=0)
    k_eff = kk_ref[0, 0, 0]
    gqx, gqy, gqz = gq_ref[0, 0, 0], gq_ref[0, 0, 1], gq_ref[0, 0, 2]

    mmdims = (((1,), (0,)), ((), ()))
    cdims = (((0,), (0,)), ((), ()))

    src3 = src_ref[0]
    tgt3 = tgt_ref[0]
    src_rot = lax.dot_general(R3_bf, src3.astype(bf16), mmdims,
                              preferred_element_type=f32)
    si3 = src_rot + c

    inner = lax.dot_general((-2.0 * si3).astype(bf16), tgt3.astype(bf16),
                            cdims, preferred_element_type=f32)
    t0, t1, t2 = tgt3[0:1, :], tgt3[1:2, :], tgt3[2:3, :]
    xx_row = xxr_ref[0]
    yy_row = t0 * t0 + t1 * t1 + t2 * t2
    dist = (jnp.transpose(xx_row) + inner) + yy_row

    iota0 = lax.broadcasted_iota(jnp.int32, (N, N), 0)
    iota1 = lax.broadcasted_iota(jnp.int32, (N, N), 1)
    BIGI = jnp.int32(N)

    minp_col = jnp.min(dist, axis=1, keepdims=True)
    idxp_col = jnp.min(jnp.where(dist == minp_col, iota1, BIGI), axis=1,
                       keepdims=True)
    minq_row = jnp.min(dist, axis=0, keepdims=True)
    idxq_row = jnp.min(jnp.where(dist == minq_row, iota0, BIGI), axis=0,
                       keepdims=True)

    tn3 = tgtn_ref[0]
    sn3 = srcn_ref[0]
    snr3 = lax.dot_general(R3_bf, sn3.astype(bf16), mmdims,
                           preferred_element_type=f32)
    inc_p_row = (snr3[0:1, :] * gqx + snr3[1:2, :] * gqy + snr3[2:3, :] * gqz)
    inc_q_row = (tn3[0:1, :] * gqx + tn3[1:2, :] * gqy + tn3[2:3, :] * gqz)

    C = N // 128

    def row_gather(vals_row, idx_row):
        er = vals_row.reshape(C, 128)
        ir = idx_row.reshape(C, 128)
        hi = ir >> 7
        lo = ir & 127
        acc = jnp.zeros((C, 128), f32)
        for c in range(C):
            bc = jnp.broadcast_to(er[c:c + 1, :], (C, 128))
            g = jnp.take_along_axis(bc, lo, axis=1)
            acc = jnp.where(hi == c, g, acc)
        return acc.reshape(1, N)

    idxp_row = jnp.transpose(idxp_col)
    inc_p_ref_row = row_gather(inc_q_row, idxp_row)
    inc_q_ref_row = row_gather(inc_p_row, idxq_row)

    minp_row = jnp.transpose(minp_col)
    nnp_row = jnp.sqrt(jnp.maximum(minp_row, 0.0) + EPS)
    nnq_row = jnp.sqrt(jnp.maximum(minq_row, 0.0) + EPS)

    K_RANK = (N - 1) // 2 + 1
    T = jnp.int32(N - K_RANK + 1)

    def _cnt(u, t):
        return jnp.sum((u >= t).astype(jnp.int32))

    def _round2(u, res, hi_bit):
        lo = hi_bit - 1
        t1 = res | jnp.int32(1 << hi_bit)
        t2 = res | jnp.int32(1 << lo)
        t3 = t1 | jnp.int32(1 << lo)
        c1, c2, c3 = _cnt(u, t1), _cnt(u, t2), _cnt(u, t3)
        hi_ok = c1 >= T
        res = jnp.where(hi_ok, t1, res)
        lo_cnt = jnp.where(hi_ok, c3, c2)
        return jnp.where(lo_cnt >= T, res | jnp.int32(1 << lo), res)

    def rank_select2(ma_row, mb_row):
        ua = lax.bitcast_convert_type(ma_row, jnp.int32).reshape(C, 128)
        ub = lax.bitcast_convert_type(mb_row, jnp.int32).reshape(C, 128)
        ta = jnp.int32(1 << 30)
        tb = jnp.int32(1 << 30)
        resa = jnp.where(_cnt(ua, ta) >= T, ta, 0)
        resb = jnp.where(_cnt(ub, tb) >= T, tb, 0)
        for hi_bit in range(29, 0, -2):
            resa = _round2(ua, resa, hi_bit)
            resb = _round2(ub, resb, hi_bit)
        return (lax.bitcast_convert_type(resa, f32),
                lax.bitcast_convert_type(resb, f32))

    mp_row = jnp.maximum(minp_row, 0.0) + EPS
    mq_row = jnp.maximum(minq_row, 0.0) + EPS
    vp, vq = rank_select2(mp_row, mq_row)
    med_p = jnp.sqrt(vp)
    med_q = jnp.sqrt(vq)

    geom_p = (nnp_row <= DIST_SCALE * med_p).astype(f32)
    geom_q = (nnq_row <= DIST_SCALE * med_q).astype(f32)

    wp = jax.nn.sigmoid(CHI2 - k_eff * (inc_p_row - inc_p_ref_row) ** 2) * geom_p
    wq = jax.nn.sigmoid(CHI2 - k_eff * (inc_q_row - inc_q_ref_row) ** 2) * geom_q
    wp_ref[0, :, :] = wp
    wq_ref[0, :, :] = wq


@jax.jit
def kernel(src, tgt, src_n, tgt_n, g_p, k_p, g_q, k_q):
    B, _, N = src.shape
    f32 = jnp.float32

    Rg = _align_rotation(g_p, g_q)
    src_rot_o = jnp.matmul(Rg, src)
    t_center = (jnp.mean(tgt, axis=2, keepdims=True)
                - jnp.mean(src_rot_o, axis=2, keepdims=True))
    src_init_o = src_rot_o + t_center
    xx_o = jnp.sum(src_init_o ** 2, axis=1, keepdims=True)
    k_eff = k_p * k_q / (k_p + k_q + EPS)

    vec_spec = pl.BlockSpec((1, 3, N), lambda b: (b, 0, 0))
    row_spec = pl.BlockSpec((1, 1, N), lambda b: (b, 0, 0))
    out_spec = pl.BlockSpec((1, 1, N), lambda b: (b, 0, 0))
    smem9 = pl.BlockSpec((1, 1, 9), lambda b: (b, 0, 0), memory_space=pltpu.SMEM)
    smem3 = pl.BlockSpec((1, 1, 3), lambda b: (b, 0, 0), memory_space=pltpu.SMEM)
    smem1 = pl.BlockSpec((1, 1, 1), lambda b: (b, 0, 0), memory_space=pltpu.SMEM)

    wp, wq = pl.pallas_call(
        functools.partial(_ght_kernel, n=N),
        grid=(B,),
        in_specs=[vec_spec, vec_spec, vec_spec, vec_spec,
                  smem9, smem3, smem1, smem3, row_spec],
        out_specs=[out_spec, out_spec],
        out_shape=[jax.ShapeDtypeStruct((B, 1, N), f32),
                   jax.ShapeDtypeStruct((B, 1, N), f32)],
        compiler_params=pltpu.CompilerParams(
            dimension_semantics=("arbitrary",),
        ),
    )(src, tgt, src_n, tgt_n,
      Rg.reshape(B, 1, 9), t_center.reshape(B, 1, 3),
      k_eff.reshape(B, 1, 1), g_q.reshape(B, 1, 3), xx_o)
    return (wp, wq)


# speedup vs baseline: 1.6512x; 1.0006x over previous
import functools
import jax
import jax.numpy as jnp
from jax import lax
from jax.experimental import pallas as pl
from jax.experimental.pallas import tpu as pltpu

EPS = 1e-06
CHI2 = 9.0
DIST_SCALE = 3.0


def _normalize_rows(x, eps=EPS):
    n = jnp.linalg.norm(x, axis=1, keepdims=True)
    return x / jnp.maximum(n, eps)


def _skew_rows(k):
    kx, ky, kz = k[:, 0], k[:, 1], k[:, 2]
    O = jnp.zeros_like(kx)
    row0 = jnp.stack([O, -kz, ky], axis=1)
    row1 = jnp.stack([kz, O, -kx], axis=1)
    row2 = jnp.stack([-ky, kx, O], axis=1)
    return jnp.stack([row0, row1, row2], axis=1)


def _align_rotation(g_src, g_tgt, eps=EPS):
    u = _normalize_rows(g_src, eps)
    v = _normalize_rows(g_tgt, eps)
    axis = jnp.cross(u, v)
    axis_norm = jnp.linalg.norm(axis, axis=1, keepdims=True)
    dot = jnp.clip(jnp.sum(u * v, axis=1, keepdims=True), -1.0, 1.0)
    parallel = axis_norm < 1e-06
    k = axis / (axis_norm + eps)
    theta = jnp.arccos(dot)
    sin_t = jnp.sin(theta)
    cos_t = jnp.cos(theta)
    K = _skew_rows(k)
    I = jnp.eye(3, dtype=g_src.dtype)[None]
    R = I + sin_t[:, :, None] * K + (1.0 - cos_t)[:, :, None] * (K @ K)
    antipar = parallel & (dot < 0.0)
    B = u.shape[0]
    ex = jnp.tile(jnp.array([1.0, 0.0, 0.0], dtype=u.dtype)[None, :], (B, 1))
    ey = jnp.tile(jnp.array([0.0, 1.0, 0.0], dtype=u.dtype)[None, :], (B, 1))
    use_ex = jnp.abs(u[:, 0:1]) < 0.9
    basis = jnp.where(use_ex, ex, ey)
    axis2 = _normalize_rows(jnp.cross(u, basis), eps)
    K2 = _skew_rows(axis2)
    R_anti = I + 2.0 * (K2 @ K2)
    R = jnp.where(antipar[:, :, None], R_anti, R)
    R = jnp.where((parallel & (dot > 0.0))[:, :, None], I, R)
    return R


def _ght_kernel(src_ref, tgt_ref, srcn_ref, tgtn_ref,
                r9_ref, c3_ref, kk_ref, gq_ref, xxr_ref,
                wp_ref, wq_ref, *, n):
    f32 = jnp.float32
    bf16 = jnp.bfloat16
    N = n

    R3 = jnp.concatenate(
        [jnp.concatenate([jnp.full((1, 1), r9_ref[0, 0, 3 * i + j], f32)
                          for j in range(3)], axis=1) for i in range(3)],
        axis=0)
    R3_bf = R3.astype(bf16)
    c = jnp.concatenate([jnp.full((1, 1), c3_ref[0, 0, i], f32)
                         for i in range(3)], axis=0)
    k_eff = kk_ref[0, 0, 0]
    gqx, gqy, gqz = gq_ref[0, 0, 0], gq_ref[0, 0, 1], gq_ref[0, 0, 2]

    mmdims = (((1,), (0,)), ((), ()))
    cdims = (((0,), (0,)), ((), ()))

    src3 = src_ref[0]
    tgt3 = tgt_ref[0]
    src_rot = lax.dot_general(R3_bf, src3.astype(bf16), mmdims,
                              preferred_element_type=f32)
    si3 = src_rot + c

    inner = lax.dot_general((-2.0 * si3).astype(bf16), tgt3.astype(bf16),
                            cdims, preferred_element_type=f32)
    t0, t1, t2 = tgt3[0:1, :], tgt3[1:2, :], tgt3[2:3, :]
    xx_row = xxr_ref[0]
    yy_row = t0 * t0 + t1 * t1 + t2 * t2
    dist = (jnp.transpose(xx_row) + inner) + yy_row

    iota0 = lax.broadcasted_iota(jnp.int32, (N, N), 0)
    iota1 = lax.broadcasted_iota(jnp.int32, (N, N), 1)
    BIGI = jnp.int32(N)

    minp_col = jnp.min(dist, axis=1, keepdims=True)
    idxp_col = jnp.min(jnp.where(dist == minp_col, iota1, BIGI), axis=1,
                       keepdims=True)
    minq_row = jnp.min(dist, axis=0, keepdims=True)
    idxq_row = jnp.min(jnp.where(dist == minq_row, iota0, BIGI), axis=0,
                       keepdims=True)

    tn3 = tgtn_ref[0]
    sn3 = srcn_ref[0]
    snr3 = lax.dot_general(R3_bf, sn3.astype(bf16), mmdims,
                           preferred_element_type=f32)
    inc_p_row = (snr3[0:1, :] * gqx + snr3[1:2, :] * gqy + snr3[2:3, :] * gqz)
    inc_q_row = (tn3[0:1, :] * gqx + tn3[1:2, :] * gqy + tn3[2:3, :] * gqz)

    C = N // 128

    def row_gather(vals_row, idx_row):
        er = vals_row.reshape(C, 128)
        ir = idx_row.reshape(C, 128)
        hi = ir >> 7
        lo = ir & 127
        acc = jnp.zeros((C, 128), f32)
        for c in range(C):
            bc = jnp.broadcast_to(er[c:c + 1, :], (C, 128))
            g = jnp.take_along_axis(bc, lo, axis=1)
            acc = jnp.where(hi == c, g, acc)
        return acc.reshape(1, N)

    idxp_row = jnp.transpose(idxp_col)
    inc_p_ref_row = row_gather(inc_q_row, idxp_row)
    inc_q_ref_row = row_gather(inc_p_row, idxq_row)

    minp_row = jnp.transpose(minp_col)
    nnp_row = jnp.sqrt(jnp.maximum(minp_row, 0.0) + EPS)
    nnq_row = jnp.sqrt(jnp.maximum(minq_row, 0.0) + EPS)

    K_RANK = (N - 1) // 2 + 1
    T = jnp.int32(N - K_RANK + 1)

    def _cnt(u, t):
        return jnp.sum((u >= t).astype(jnp.int32))

    def _round2(u, res, hi_bit):
        lo = hi_bit - 1
        t1 = res | jnp.int32(1 << hi_bit)
        t2 = res | jnp.int32(1 << lo)
        t3 = t1 | jnp.int32(1 << lo)
        c1, c2, c3 = _cnt(u, t1), _cnt(u, t2), _cnt(u, t3)
        hi_ok = c1 >= T
        res = jnp.where(hi_ok, t1, res)
        lo_cnt = jnp.where(hi_ok, c3, c2)
        return jnp.where(lo_cnt >= T, res | jnp.int32(1 << lo), res)

    def rank_select2(ma_row, mb_row):
        ua = lax.bitcast_convert_type(ma_row, jnp.int32).reshape(C, 128)
        ub = lax.bitcast_convert_type(mb_row, jnp.int32).reshape(C, 128)
        ta = jnp.int32(1 << 30)
        tb = jnp.int32(1 << 30)
        resa = jnp.where(_cnt(ua, ta) >= T, ta, 0)
        resb = jnp.where(_cnt(ub, tb) >= T, tb, 0)
        for hi_bit in range(29, 0, -2):
            resa = _round2(ua, resa, hi_bit)
            resb = _round2(ub, resb, hi_bit)
        return (lax.bitcast_convert_type(resa, f32),
                lax.bitcast_convert_type(resb, f32))

    mp_row = jnp.maximum(minp_row, 0.0) + EPS
    mq_row = jnp.maximum(minq_row, 0.0) + EPS
    vp, vq = rank_select2(mp_row, mq_row)
    med_p = jnp.sqrt(vp)
    med_q = jnp.sqrt(vq)

    geom_p = (nnp_row <= DIST_SCALE * med_p).astype(f32)
    geom_q = (nnq_row <= DIST_SCALE * med_q).astype(f32)

    wp = jax.nn.sigmoid(CHI2 - k_eff * (inc_p_row - inc_p_ref_row) ** 2) * geom_p
    wq = jax.nn.sigmoid(CHI2 - k_eff * (inc_q_row - inc_q_ref_row) ** 2) * geom_q
    wp_ref[0, :, :] = wp
    wq_ref[0, :, :] = wq


@jax.jit
def kernel(src, tgt, src_n, tgt_n, g_p, k_p, g_q, k_q):
    B, _, N = src.shape
    f32 = jnp.float32

    Rg = _align_rotation(g_p, g_q)
    src_rot_o = jnp.matmul(Rg, src)
    t_center = (jnp.mean(tgt, axis=2, keepdims=True)
                - jnp.mean(src_rot_o, axis=2, keepdims=True))
    src_init_o = src_rot_o + t_center
    xx_o = jnp.sum(src_init_o ** 2, axis=1, keepdims=True)
    k_eff = k_p * k_q / (k_p + k_q + EPS)

    vec_spec = pl.BlockSpec((1, 3, N), lambda b: (b, 0, 0))
    row_spec = pl.BlockSpec((1, 1, N), lambda b: (b, 0, 0))
    out_spec = pl.BlockSpec((1, 1, N), lambda b: (b, 0, 0))
    smem9 = pl.BlockSpec((1, 1, 9), lambda b: (b, 0, 0), memory_space=pltpu.SMEM)
    smem3 = pl.BlockSpec((1, 1, 3), lambda b: (b, 0, 0), memory_space=pltpu.SMEM)
    smem1 = pl.BlockSpec((1, 1, 1), lambda b: (b, 0, 0), memory_space=pltpu.SMEM)

    wp, wq = pl.pallas_call(
        functools.partial(_ght_kernel, n=N),
        grid=(B,),
        in_specs=[vec_spec, vec_spec, vec_spec, vec_spec,
                  smem9, smem3, smem1, smem3, row_spec],
        out_specs=[out_spec, out_spec],
        out_shape=[jax.ShapeDtypeStruct((B, 1, N), f32),
                   jax.ShapeDtypeStruct((B, 1, N), f32)],
        compiler_params=pltpu.CompilerParams(
            dimension_semantics=("parallel",),
        ),
    )(src, tgt, src_n, tgt_n,
      Rg.reshape(B, 1, 9), t_center.reshape(B, 1, 3),
      k_eff.reshape(B, 1, 1), g_q.reshape(B, 1, 3), xx_o)
    return (wp, wq)
